# Initial kernel scaffold; baseline (speedup 1.0000x reference)
import jax
import jax.numpy as jnp
from jax.experimental import pallas as pl


def kernel(data, table, W1, b1, W2, b2):
    raise NotImplementedError("write your pallas kernel here")


# speedup vs baseline: 1.8332x; 1.8332x over previous
import functools

import jax
import jax.numpy as jnp
from jax import lax
from jax.experimental import pallas as pl
from jax.experimental.pallas import tpu as pltpu
from jax.experimental.pallas import tpu_sc as plsc

VOCAB = 1000000
EMBED = 64
HIDDEN = 256
NCLASS = 16
BATCH = 16384
HIST = 50

_NC = 2
_NS = 16
NW = _NC * _NS
BPW = BATCH // NW
LANES = 16
EV = EMBED // LANES


def _sc_pooled(table, data3):
    mesh = plsc.VectorSubcoreMesh(core_axis_name="c", subcore_axis_name="s")

    @functools.partial(
        pl.kernel,
        mesh=mesh,
        out_type=jax.ShapeDtypeStruct((NW, BPW, EMBED), jnp.float32),
        scratch_types=[
            pltpu.VMEM((BPW, HIST), jnp.int32),
            pltpu.VMEM((HIST, EMBED), jnp.float32),
            pltpu.VMEM((BPW, EMBED), jnp.float32),
            pltpu.SemaphoreType.DMA,
        ],
        compiler_params=pltpu.CompilerParams(use_tc_tiling_on_sc=False),
    )
    def sc_kernel(table_hbm, data_hbm, out_hbm, idx_v, rows_v, pooled_v, sem):
        wid = lax.axis_index("s") * _NC + lax.axis_index("c")
        pltpu.sync_copy(data_hbm.at[wid], idx_v)

        def bag(g, _):
            pltpu.async_copy(table_hbm.at[idx_v.at[g]], rows_v, sem).wait()

            def red(j, accs):
                return tuple(
                    a + rows_v[j, pl.ds(LANES * k, LANES)]
                    for k, a in enumerate(accs)
                )

            accs = tuple(rows_v[0, pl.ds(LANES * k, LANES)] for k in range(EV))
            accs = lax.fori_loop(1, HIST, red, accs)
            for k in range(EV):
                pooled_v[g, pl.ds(LANES * k, LANES)] = accs[k] * (1.0 / HIST)
            return 0

        lax.fori_loop(0, BPW, bag, 0)
        pltpu.sync_copy(pooled_v, out_hbm.at[wid])

    return sc_kernel(table, data3)


def _tc_mlp(pooled, W1, b1, W2, b2):
    BLK = 1024

    def body(x_ref, w1_ref, b1_ref, w2_ref, b2_ref, o_ref):
        x = x_ref[...]
        h = jnp.dot(x, w1_ref[...], preferred_element_type=jnp.float32) + b1_ref[...]
        l = jnp.dot(h, w2_ref[...], preferred_element_type=jnp.float32) + b2_ref[...]
        m = jnp.max(l, axis=-1, keepdims=True)
        e = jnp.exp(l - m)
        o_ref[...] = e / jnp.sum(e, axis=-1, keepdims=True)

    return pl.pallas_call(
        body,
        grid=(BATCH // BLK,),
        in_specs=[
            pl.BlockSpec((BLK, EMBED), lambda i: (i, 0)),
            pl.BlockSpec((EMBED, HIDDEN), lambda i: (0, 0)),
            pl.BlockSpec((1, HIDDEN), lambda i: (0, 0)),
            pl.BlockSpec((HIDDEN, NCLASS), lambda i: (0, 0)),
            pl.BlockSpec((1, NCLASS), lambda i: (0, 0)),
        ],
        out_specs=pl.BlockSpec((BLK, NCLASS), lambda i: (i, 0)),
        out_shape=jax.ShapeDtypeStruct((BATCH, NCLASS), jnp.float32),
    )(pooled, W1, b1.reshape(1, HIDDEN), W2, b2.reshape(1, NCLASS))


def kernel(data, table, W1, b1, W2, b2):
    data3 = data.astype(jnp.int32).reshape(NW, BPW, HIST)
    pooled = _sc_pooled(table, data3).reshape(BATCH, EMBED)
    return _tc_mlp(pooled, W1, b1, W2, b2)


# speedup vs baseline: 2.5174x; 1.3732x over previous
import functools

import jax
import jax.numpy as jnp
from jax import lax
from jax.experimental import pallas as pl
from jax.experimental.pallas import tpu as pltpu
from jax.experimental.pallas import tpu_sc as plsc

VOCAB = 1000000
EMBED = 64
HIDDEN = 256
NCLASS = 16
BATCH = 16384
HIST = 50

_NC = 2
_NS = 16
NW = _NC * _NS
BPW = BATCH // NW
LANES = 16
EV = EMBED // LANES


BAGS_PER_DMA = 2
NPAIR = BPW // BAGS_PER_DMA
NBUF = 2


def _sc_pooled(table, data3):
    mesh = plsc.VectorSubcoreMesh(core_axis_name="c", subcore_axis_name="s")
    GROUP = BAGS_PER_DMA * HIST

    @functools.partial(
        pl.kernel,
        mesh=mesh,
        out_type=jax.ShapeDtypeStruct((NW, BPW, EMBED), jnp.float32),
        scratch_types=[
            pltpu.VMEM((NPAIR, GROUP), jnp.int32),
            pltpu.VMEM((NBUF, GROUP, EMBED), jnp.float32),
            pltpu.VMEM((BPW, EMBED), jnp.float32),
            [pltpu.SemaphoreType.DMA] * NBUF,
        ],
        compiler_params=pltpu.CompilerParams(use_tc_tiling_on_sc=False),
    )
    def sc_kernel(table_hbm, data_hbm, out_hbm, idx_v, rows_v, pooled_v, sems):
        wid = lax.axis_index("s") * _NC + lax.axis_index("c")
        pltpu.sync_copy(data_hbm.at[wid], idx_v)

        for b in range(NBUF):
            pltpu.async_copy(table_hbm.at[idx_v.at[b]], rows_v.at[b], sems[b])

        def group_body(i, _):
            p0 = i * NBUF
            for b in range(NBUF):
                p = p0 + b
                buf = rows_v.at[b]
                pltpu.make_async_copy(table_hbm.at[idx_v.at[p]], buf, sems[b]).wait()
                for bag in range(BAGS_PER_DMA):
                    base = bag * HIST
                    accs = [buf[base, pl.ds(LANES * k, LANES)] for k in range(EV)]
                    for j in range(1, HIST):
                        for k in range(EV):
                            accs[k] = accs[k] + buf[base + j, pl.ds(LANES * k, LANES)]
                    for k in range(EV):
                        pooled_v[p * BAGS_PER_DMA + bag, pl.ds(LANES * k, LANES)] = (
                            accs[k] * (1.0 / HIST)
                        )
                nxt = p + NBUF

                @pl.when(nxt < NPAIR)
                def _():
                    pltpu.async_copy(table_hbm.at[idx_v.at[nxt]], buf, sems[b])

            return 0

        lax.fori_loop(0, NPAIR // NBUF, group_body, 0)
        pltpu.sync_copy(pooled_v, out_hbm.at[wid])

    return sc_kernel(table, data3)


def _tc_mlp(pooled, W1, b1, W2, b2):
    BLK = 1024

    def body(x_ref, w1_ref, b1_ref, w2_ref, b2_ref, o_ref):
        x = x_ref[...]
        h = jnp.dot(x, w1_ref[...], preferred_element_type=jnp.float32) + b1_ref[...]
        l = jnp.dot(h, w2_ref[...], preferred_element_type=jnp.float32) + b2_ref[...]
        m = jnp.max(l, axis=-1, keepdims=True)
        e = jnp.exp(l - m)
        o_ref[...] = e / jnp.sum(e, axis=-1, keepdims=True)

    return pl.pallas_call(
        body,
        grid=(BATCH // BLK,),
        in_specs=[
            pl.BlockSpec((BLK, EMBED), lambda i: (i, 0)),
            pl.BlockSpec((EMBED, HIDDEN), lambda i: (0, 0)),
            pl.BlockSpec((1, HIDDEN), lambda i: (0, 0)),
            pl.BlockSpec((HIDDEN, NCLASS), lambda i: (0, 0)),
            pl.BlockSpec((1, NCLASS), lambda i: (0, 0)),
        ],
        out_specs=pl.BlockSpec((BLK, NCLASS), lambda i: (i, 0)),
        out_shape=jax.ShapeDtypeStruct((BATCH, NCLASS), jnp.float32),
    )(pooled, W1, b1.reshape(1, HIDDEN), W2, b2.reshape(1, NCLASS))


def kernel(data, table, W1, b1, W2, b2):
    data3 = data.astype(jnp.int32).reshape(NW, NPAIR, BAGS_PER_DMA * HIST)
    pooled = _sc_pooled(table, data3).reshape(BATCH, EMBED)
    return _tc_mlp(pooled, W1, b1, W2, b2)


# speedup vs baseline: 4.4601x; 1.7717x over previous
import functools

import jax
import jax.numpy as jnp
from jax import lax
from jax.experimental import pallas as pl
from jax.experimental.pallas import tpu as pltpu
from jax.experimental.pallas import tpu_sc as plsc

VOCAB = 1000000
EMBED = 64
HIDDEN = 256
NCLASS = 16
BATCH = 16384
HIST = 50

_NC = 2
_NS = 16
NW = _NC * _NS
BPW = BATCH // NW
LANES = 16
EV = EMBED // LANES


BAGS_PER_DMA = 2
NPAIR = BPW // BAGS_PER_DMA
NBUF = 2


def _sc_pooled(table, data3):
    mesh = plsc.VectorSubcoreMesh(core_axis_name="c", subcore_axis_name="s")
    GROUP = BAGS_PER_DMA * HIST

    @functools.partial(
        pl.kernel,
        mesh=mesh,
        out_type=jax.ShapeDtypeStruct((NW, BPW, EMBED), jnp.float32),
        scratch_types=[
            pltpu.VMEM((NPAIR, GROUP), jnp.int32),
            pltpu.VMEM((NBUF, GROUP, EMBED), jnp.float32),
            pltpu.VMEM((BPW, EMBED), jnp.float32),
            [pltpu.SemaphoreType.DMA] * NBUF,
        ],
        compiler_params=pltpu.CompilerParams(use_tc_tiling_on_sc=False),
    )
    def sc_kernel(table_hbm, data_hbm, out_hbm, idx_v, rows_v, pooled_v, sems):
        wid = lax.axis_index("s") * _NC + lax.axis_index("c")
        pltpu.sync_copy(data_hbm.at[wid], idx_v)

        for b in range(NBUF):
            pltpu.async_copy(table_hbm.at[idx_v.at[b]], rows_v.at[b], sems[b])

        def group_body(i, _):
            p0 = i * NBUF
            for b in range(NBUF):
                p = p0 + b
                buf = rows_v.at[b]
                pltpu.make_async_copy(table_hbm.at[idx_v.at[p]], buf, sems[b]).wait()
                for bag in range(BAGS_PER_DMA):
                    base = bag * HIST
                    accs = [buf[base, pl.ds(LANES * k, LANES)] for k in range(EV)]
                    for j in range(1, HIST):
                        for k in range(EV):
                            accs[k] = accs[k] + buf[base + j, pl.ds(LANES * k, LANES)]
                    for k in range(EV):
                        pooled_v[p * BAGS_PER_DMA + bag, pl.ds(LANES * k, LANES)] = (
                            accs[k] * (1.0 / HIST)
                        )
                nxt = p + NBUF

                @pl.when(nxt < NPAIR)
                def _():
                    pltpu.async_copy(table_hbm.at[idx_v.at[nxt]], buf, sems[b])

            return 0

        lax.fori_loop(0, NPAIR // NBUF, group_body, 0)
        pltpu.sync_copy(pooled_v, out_hbm.at[wid])

    return sc_kernel(table, data3)


def _tc_mlp(pooled, W1, b1, W2, b2):
    BLK = 1024

    def body(x_ref, w1_ref, b1_ref, w2_ref, b2_ref, o_ref):
        x = x_ref[...]
        h = jnp.dot(x, w1_ref[...], preferred_element_type=jnp.float32) + b1_ref[...]
        l = jnp.dot(h, w2_ref[...], preferred_element_type=jnp.float32) + b2_ref[...]
        m = jnp.max(l, axis=-1, keepdims=True)
        e = jnp.exp(l - m)
        o_ref[...] = e / jnp.sum(e, axis=-1, keepdims=True)

    return pl.pallas_call(
        body,
        grid=(BATCH // BLK,),
        in_specs=[
            pl.BlockSpec((BLK, EMBED), lambda i: (i, 0)),
            pl.BlockSpec((EMBED, HIDDEN), lambda i: (0, 0)),
            pl.BlockSpec((1, HIDDEN), lambda i: (0, 0)),
            pl.BlockSpec((HIDDEN, NCLASS), lambda i: (0, 0)),
            pl.BlockSpec((1, NCLASS), lambda i: (0, 0)),
        ],
        out_specs=pl.BlockSpec((BLK, NCLASS), lambda i: (i, 0)),
        out_shape=jax.ShapeDtypeStruct((BATCH, NCLASS), jnp.float32),
    )(pooled, W1, b1.reshape(1, HIDDEN), W2, b2.reshape(1, NCLASS))


VB = 16384
VGRID = -(-VOCAB // VB)
VPAD = VGRID * VB


def _tc_detile(tableT):
    def body(x_ref, o_ref):
        x = x_ref[...]
        o_ref[:, :EMBED] = x[:, : VB // 2].T
        o_ref[:, EMBED:] = x[:, VB // 2 :].T

    return pl.pallas_call(
        body,
        grid=(VGRID,),
        in_specs=[pl.BlockSpec((EMBED, VB), lambda i: (0, i))],
        out_specs=pl.BlockSpec((VB // 2, 2 * EMBED), lambda i: (i, 0)),
        out_shape=jax.ShapeDtypeStruct((VPAD // 2, 2 * EMBED), jnp.float32),
    )(tableT)


def kernel(data, table, W1, b1, W2, b2):
    v = data.astype(jnp.int32)
    r = v & (VB - 1)
    vmap = (v - r) + ((r & (VB // 2 - 1)) << 1) + (r >> 13)
    data3 = vmap.reshape(NW, NPAIR, BAGS_PER_DMA * HIST)
    table_lin = _tc_detile(table.T).reshape(VPAD, EMBED)
    pooled = _sc_pooled(table_lin, data3).reshape(BATCH, EMBED)
    return _tc_mlp(pooled, W1, b1, W2, b2)
